# Initial kernel scaffold; baseline (speedup 1.0000x reference)
import jax
import jax.numpy as jnp
from jax.experimental import pallas as pl


def kernel(x, dR, neighbors_idx, pairwise_mask, dR_expanded, W_in2f, W_f1, b_f1, W_f2, b_f2, W_out, b_out):
    raise NotImplementedError("write your pallas kernel here")


# speedup vs baseline: 1584.0082x; 1584.0082x over previous
import functools

import jax
import jax.numpy as jnp
from jax import lax
from jax.experimental import pallas as pl
from jax.experimental.pallas import tpu as pltpu
from jax.experimental.pallas import tpu_sc as plsc

N, NBH, N_IN, NF, NG = 10000, 32, 128, 128, 25
E_TOT = N * NBH
CUTOFF = 5.0
LN2 = 0.6931471805599453

T = 200
ET = T * NBH
GRID = N // T

NW = 32
EPW = E_TOT // NW
CH = 400
NCH = EPW // CH


def _ssp(v):
    return jnp.maximum(v, 0.0) + jnp.log1p(jnp.exp(-jnp.abs(v))) - LN2


def _pre_body(x_ref, w_ref, dr_ref, mask_ref, y_ref, cm_ref):
    y_ref[...] = jnp.dot(x_ref[...], w_ref[...],
                         preferred_element_type=jnp.float32)
    dr = dr_ref[...]
    c = 0.5 * (jnp.cos(dr * (jnp.pi / CUTOFF)) + 1.0)
    c = c * (dr < CUTOFF).astype(jnp.float32)
    cm_ref[...] = c * mask_ref[...]


def _pre(x, w_in2f, dr_col, mask_col):
    return pl.pallas_call(
        _pre_body,
        grid=(GRID,),
        in_specs=[
            pl.BlockSpec((T, N_IN), lambda i: (i, 0)),
            pl.BlockSpec((N_IN, NF), lambda i: (0, 0)),
            pl.BlockSpec((ET, 1), lambda i: (i, 0)),
            pl.BlockSpec((ET, 1), lambda i: (i, 0)),
        ],
        out_specs=[
            pl.BlockSpec((T, NF), lambda i: (i, 0)),
            pl.BlockSpec((ET, 1), lambda i: (i, 0)),
        ],
        out_shape=[
            jax.ShapeDtypeStruct((N, NF), jnp.float32),
            jax.ShapeDtypeStruct((E_TOT, 1), jnp.float32),
        ],
    )(x, w_in2f, dr_col, mask_col)


def _gather(y, idx_flat):
    mesh = plsc.VectorSubcoreMesh(core_axis_name="c", subcore_axis_name="s")

    @functools.partial(
        pl.kernel,
        out_type=jax.ShapeDtypeStruct((E_TOT, NF), jnp.float32),
        mesh=mesh,
        scratch_types=[
            pltpu.VMEM((CH,), jnp.int32),
            pltpu.VMEM((CH, NF), jnp.float32),
            pltpu.SemaphoreType.DMA,
        ],
    )
    def k(ytab_hbm, idx_hbm, out_hbm, idx_v, rows_v, sem):
        nc = mesh.num_cores
        wid = lax.axis_index("s") * nc + lax.axis_index("c")
        base = wid * EPW

        def body(c, carry):
            off = base + c * CH
            pltpu.sync_copy(idx_hbm.at[pl.ds(off, CH)], idx_v)
            pltpu.async_copy(ytab_hbm.at[idx_v], rows_v, sem).wait()
            pltpu.sync_copy(rows_v, out_hbm.at[pl.ds(off, CH)])
            return carry

        lax.fori_loop(0, NCH, body, 0)

    return k(y, idx_flat)


def _main_body(a_ref, yg_ref, cm_ref, wf1_ref, bf1_ref, wf2_ref, bf2_ref,
               wout_ref, bout_ref, o_ref):
    h = _ssp(jnp.dot(a_ref[...], wf1_ref[...],
                     preferred_element_type=jnp.float32) + bf1_ref[...])
    w = jnp.dot(h, wf2_ref[...],
                preferred_element_type=jnp.float32) + bf2_ref[...]
    z = w * cm_ref[...] * yg_ref[...]
    s = jnp.sum(z.reshape(T, NBH, NF), axis=1)
    o_ref[...] = _ssp(jnp.dot(s, wout_ref[...],
                              preferred_element_type=jnp.float32)
                      + bout_ref[...])


def _main(a2d, yg, cm, w_f1, b_f1, w_f2, b_f2, w_out, b_out):
    full = lambda r, c: pl.BlockSpec((r, c), lambda i: (0, 0))
    return pl.pallas_call(
        _main_body,
        grid=(GRID,),
        in_specs=[
            pl.BlockSpec((ET, NG), lambda i: (i, 0)),
            pl.BlockSpec((ET, NF), lambda i: (i, 0)),
            pl.BlockSpec((ET, 1), lambda i: (i, 0)),
            full(NG, NF), full(1, NF), full(NF, NF), full(1, NF),
            full(NF, NF), full(1, NF),
        ],
        out_specs=pl.BlockSpec((T, NF), lambda i: (i, 0)),
        out_shape=jax.ShapeDtypeStruct((N, NF), jnp.float32),
    )(a2d, yg, cm, w_f1, b_f1, w_f2, b_f2, w_out, b_out)


def kernel(x, dR, neighbors_idx, pairwise_mask, dR_expanded,
           W_in2f, W_f1, b_f1, W_f2, b_f2, W_out, b_out):
    dr_col = dR.reshape(E_TOT, 1)
    mask_col = pairwise_mask.reshape(E_TOT, 1)
    a2d = dR_expanded.reshape(E_TOT, NG)
    idx_flat = neighbors_idx.reshape(E_TOT).astype(jnp.int32)

    y, cm = _pre(x, W_in2f, dr_col, mask_col)
    yg = _gather(y, idx_flat)
    return _main(a2d, yg, cm, W_f1, b_f1.reshape(1, NF), W_f2,
                 b_f2.reshape(1, NF), W_out, b_out.reshape(1, NF))


# speedup vs baseline: 4546.8764x; 2.8705x over previous
import functools

import jax
import jax.numpy as jnp
from jax import lax
from jax.experimental import pallas as pl
from jax.experimental.pallas import tpu as pltpu
from jax.experimental.pallas import tpu_sc as plsc

N, NBH, N_IN, NF, NG = 10000, 32, 128, 128, 25
E_TOT = N * NBH
CUTOFF = 5.0
LN2 = 0.6931471805599453
LOG2E = 1.4426950408889634

T = 200
ET = T * NBH
GRID = N // T

TP = 1000
GRID_PRE = N // TP
EPK = E_TOT // 128

NW = 32
EPW = E_TOT // NW
CH = 200
NCH = EPW // CH
NCH2 = NCH // 2


def _exp2_log2p1(v):
    u = jnp.exp2(jnp.minimum(v, 126.0))
    return jnp.log2(1.0 + u)


def _pre_body(x_ref, w_ref, y_ref):
    y_ref[...] = jnp.dot(x_ref[...], w_ref[...],
                         preferred_element_type=jnp.float32)


def _pre(x, w_in2f):
    return pl.pallas_call(
        _pre_body,
        grid=(GRID_PRE,),
        in_specs=[
            pl.BlockSpec((TP, N_IN), lambda i: (i, 0)),
            pl.BlockSpec((N_IN, NF), lambda i: (0, 0)),
        ],
        out_specs=pl.BlockSpec((TP, NF), lambda i: (i, 0)),
        out_shape=jax.ShapeDtypeStruct((N, NF), jnp.float32),
    )(x, w_in2f)


def _cut_body(dr_ref, mask_ref, cm_ref):
    dr = dr_ref[...]
    c = 0.5 * (jnp.cos(dr * (jnp.pi / CUTOFF)) + 1.0)
    c = c * (dr < CUTOFF).astype(jnp.float32)
    cm_ref[...] = c * mask_ref[...]


def _cut(dr_pk, mask_pk):
    return pl.pallas_call(
        _cut_body,
        out_shape=jax.ShapeDtypeStruct((EPK, 128), jnp.float32),
    )(dr_pk, mask_pk)


def _gather(y, idx_flat):
    mesh = plsc.VectorSubcoreMesh(core_axis_name="c", subcore_axis_name="s")

    @functools.partial(
        pl.kernel,
        out_type=jax.ShapeDtypeStruct((E_TOT, NF), jnp.float32),
        mesh=mesh,
        scratch_types=[
            pltpu.VMEM((CH,), jnp.int32),
            pltpu.VMEM((CH,), jnp.int32),
            pltpu.VMEM((CH, NF), jnp.float32),
            pltpu.VMEM((CH, NF), jnp.float32),
            pltpu.SemaphoreType.DMA,
            pltpu.SemaphoreType.DMA,
            pltpu.SemaphoreType.DMA,
            pltpu.SemaphoreType.DMA,
            pltpu.SemaphoreType.DMA,
            pltpu.SemaphoreType.DMA,
        ],
    )
    def k(ytab, idx_hbm, out_hbm, i0, i1, r0, r1,
          si0, si1, sg0, sg1, sw0, sw1):
        nc = mesh.num_cores
        wid = lax.axis_index("s") * nc + lax.axis_index("c")
        base = wid * EPW
        ib, rb = (i0, i1), (r0, r1)
        si, sg, sw = (si0, si1), (sg0, sg1), (sw0, sw1)

        def idx_cp(c, b):
            return pltpu.make_async_copy(
                idx_hbm.at[pl.ds(base + c * CH, CH)], ib[b], si[b])

        def gath(b):
            return pltpu.make_async_copy(ytab.at[ib[b]], rb[b], sg[b])

        def wrb(c, b):
            return pltpu.make_async_copy(
                rb[b], out_hbm.at[pl.ds(base + c * CH, CH)], sw[b])

        idx_cp(0, 0).start()
        idx_cp(1, 1).start()

        def body(c2, carry):
            c0 = c2 * 2
            c1 = c0 + 1

            @pl.when(c2 > 0)
            def _():
                wrb(c0 - 2, 0).wait()
                wrb(c1 - 2, 1).wait()

            idx_cp(c0, 0).wait()
            gath(0).start()
            idx_cp(c1, 1).wait()
            gath(0).wait()
            wrb(c0, 0).start()
            gath(1).start()

            @pl.when(c2 < NCH2 - 1)
            def _():
                idx_cp(c0 + 2, 0).start()

            gath(1).wait()
            wrb(c1, 1).start()

            @pl.when(c2 < NCH2 - 1)
            def _():
                idx_cp(c1 + 2, 1).start()

            return carry

        lax.fori_loop(0, NCH2, body, 0)
        wrb(NCH - 2, 0).wait()
        wrb(NCH - 1, 1).wait()

    return k(y, idx_flat)


def _main_body(a_ref, yg_ref, cm_ref, wf1_ref, bf1_ref, wf2_ref, bf2_ref,
               wout_ref, bout_ref, o_ref):
    vp = jnp.dot(a_ref[...], wf1_ref[...],
                 preferred_element_type=jnp.float32) + bf1_ref[...]
    h = _exp2_log2p1(vp)
    w = jnp.dot(h, wf2_ref[...],
                preferred_element_type=jnp.float32) + bf2_ref[...]
    z = w * cm_ref[...] * yg_ref[...]
    s = jnp.sum(z.reshape(T, NBH, NF), axis=1)
    op = jnp.dot(s, wout_ref[...],
                 preferred_element_type=jnp.float32) + bout_ref[...]
    o_ref[...] = LN2 * _exp2_log2p1(op) - LN2


def _main(a2d, yg, cm, w_f1, b_f1, w_f2, b_f2, w_out, b_out):
    full = lambda r, c: pl.BlockSpec((r, c), lambda i: (0, 0))
    return pl.pallas_call(
        _main_body,
        grid=(GRID,),
        in_specs=[
            pl.BlockSpec((ET, NG), lambda i: (i, 0)),
            pl.BlockSpec((ET, NF), lambda i: (i, 0)),
            pl.BlockSpec((ET, 1), lambda i: (i, 0)),
            full(NG, NF), full(1, NF), full(NF, NF), full(1, NF),
            full(NF, NF), full(1, NF),
        ],
        out_specs=pl.BlockSpec((T, NF), lambda i: (i, 0)),
        out_shape=jax.ShapeDtypeStruct((N, NF), jnp.float32),
    )(a2d, yg, cm, w_f1, b_f1, w_f2, b_f2, w_out, b_out)


def kernel(x, dR, neighbors_idx, pairwise_mask, dR_expanded,
           W_in2f, W_f1, b_f1, W_f2, b_f2, W_out, b_out):
    dr_pk = dR.reshape(E_TOT // 128, 128)
    mask_pk = pairwise_mask.reshape(E_TOT // 128, 128)
    a2d = dR_expanded.reshape(E_TOT, NG)
    idx_flat = neighbors_idx.reshape(E_TOT).astype(jnp.int32)

    wf1 = W_f1 * LOG2E
    bf1 = (b_f1 * LOG2E).reshape(1, NF)
    wf2 = W_f2 * LN2
    bf2 = (b_f2 - LN2 * jnp.sum(W_f2, axis=0)).reshape(1, NF)
    wout = W_out * LOG2E
    bout = (b_out * LOG2E).reshape(1, NF)

    y = _pre(x, W_in2f)
    cm = _cut(dr_pk, mask_pk)
    yg = _gather(y, idx_flat)
    cm_col = cm.reshape(E_TOT, 1)
    return _main(a2d, yg, cm_col, wf1, bf1, wf2, bf2, wout, bout)
